# baseline (speedup 1.0000x reference)
import functools

import jax
import jax.numpy as jnp
from jax import lax
from jax.experimental import pallas as pl
from jax.experimental.pallas import tpu as pltpu
from jax.experimental.pallas import tpu_sc as plsc

_NUM_USERS = 1000000
_DIM = 16
_BATCH = 16384
_NC = 2
_NS = 16
_NW = _NC * _NS
_BPW = _BATCH // _NW
_CHUNK = 128
_NCH = _BPW // _CHUNK

_LC = 2048
_NFULL = _NUM_USERS // _LC
_ALIGNED_END = (_NUM_USERS // 128) * 128
_TAIL = _ALIGNED_END - _NFULL * _LC
_RAG = _NUM_USERS - _ALIGNED_END
_NTC = -(-_NUM_USERS // 128)
_ROWS = _NTC * _DIM

_mesh = plsc.VectorSubcoreMesh(core_axis_name="c", subcore_axis_name="s")


@functools.partial(
    pl.kernel,
    mesh=_mesh,
    out_type=jax.ShapeDtypeStruct((_ROWS, 128), jnp.float32),
    scratch_types=[
        pltpu.VMEM((2, 16 * _DIM, 128), jnp.float32),
        pltpu.VMEM((4 * _DIM, 128), jnp.float32),
        pltpu.VMEM((_DIM, 128), jnp.float32),
        pltpu.SemaphoreType.DMA,
        pltpu.SemaphoreType.DMA,
    ],
)
def _repack(w_hbm, wtail_hbm, wlin_hbm, ring, tailbuf, wt, sem_in, sem_out):
    wid = lax.axis_index("s") * _NC + lax.axis_index("c")

    def body(jj, carry):
        tc = wid + jj * _NW

        @pl.when(tc < _NTC - 1)
        def _():
            c0 = pl.multiple_of(tc * 128, 128)
            pltpu.async_copy(
                w_hbm.at[:, pl.ds(c0, 128)],
                wlin_hbm.at[pl.ds(tc * _DIM, _DIM), :],
                sem_out,
            )

        return carry

    nfj = (_NTC - 1) // _NW + 1
    lax.fori_loop(0, nfj, body, 0)

    def drain(jj, carry):
        tc = wid + jj * _NW

        @pl.when(tc < _NTC - 1)
        def _():
            pltpu.make_async_copy(
                w_hbm.at[:, pl.ds(0, 128)],
                wlin_hbm.at[pl.ds(0, _DIM), :],
                sem_out,
            ).wait()

        return carry

    lax.fori_loop(0, nfj, drain, 0)

    @pl.when(wid == 0)
    def _():
        pltpu.sync_copy(wtail_hbm, wt)
        pltpu.sync_copy(wt, wlin_hbm.at[pl.ds((_NTC - 1) * _DIM, _DIM), :])


@functools.partial(
    pl.kernel,
    mesh=_mesh,
    out_type=jax.ShapeDtypeStruct((_DIM, _BATCH), jnp.float32),
    scratch_types=[
        pltpu.VMEM((_BPW,), jnp.int32),
        pltpu.VMEM((_DIM, _BPW), jnp.int32),
        pltpu.VMEM((_DIM, _BPW), jnp.float32),
        pltpu.SemaphoreType.DMA,
    ],
)
def _lookup(wlin_hbm, x_hbm, out_hbm, xv, idxv, rowsd, sem):
    wid = lax.axis_index("s") * _NC + lax.axis_index("c")
    base = wid * _BPW
    pltpu.sync_copy(x_hbm.at[pl.ds(base, _BPW)], xv)

    def offsets(c, carry):
        vx = xv[pl.ds(c * _NS, _NS)]
        vt = (vx >> 7) * 2048 + (vx & 127)
        for d in range(_DIM):
            idxv[d, pl.ds(c * _NS, _NS)] = vt + ((d // 8) * 1024 + (d % 8) * 128)
        return carry

    lax.fori_loop(0, _BPW // _NS, offsets, 0)

    copies = [
        pltpu.async_copy(
            wlin_hbm.at[idxv.at[d, pl.ds(k * _CHUNK, _CHUNK)]],
            rowsd.at[d, pl.ds(k * _CHUNK, _CHUNK)],
            sem,
        )
        for d in range(_DIM)
        for k in range(_NCH)
    ]
    for cp in copies:
        cp.wait()

    pltpu.sync_copy(rowsd, out_hbm.at[:, pl.ds(base, _BPW)])


def kernel(x, W):
    wtail = jnp.pad(W[:, _ALIGNED_END:], ((0, 0), (0, 128 - _RAG)))
    wlin = _repack(W, wtail).reshape(-1)
    h = _lookup(wlin, x.astype(jnp.int32))
    return h.T


# speedup vs baseline: 23.8524x; 23.8524x over previous
import functools

import jax
import jax.numpy as jnp
from jax import lax
from jax.experimental import pallas as pl
from jax.experimental.pallas import tpu as pltpu
from jax.experimental.pallas import tpu_sc as plsc

_NUM_USERS = 1000000
_DIM = 16
_BATCH = 16384
_NC = 2
_NS = 16
_NW = _NC * _NS
_BPW = _BATCH // _NW
_CHUNK = 128
_NCH = _BPW // _CHUNK

_LC = 2048
_NFULL = _NUM_USERS // _LC
_ALIGNED_END = (_NUM_USERS // 128) * 128
_TAIL = _ALIGNED_END - _NFULL * _LC
_RAG = _NUM_USERS - _ALIGNED_END
_NTC = -(-_NUM_USERS // 128)
_ROWS = _NTC * _DIM

_mesh = plsc.VectorSubcoreMesh(core_axis_name="c", subcore_axis_name="s")


@functools.partial(
    pl.kernel,
    mesh=_mesh,
    out_type=jax.ShapeDtypeStruct((_ROWS, 128), jnp.float32),
    scratch_types=[
        pltpu.VMEM((3, 16 * _DIM, 128), jnp.float32),
        pltpu.VMEM((4 * _DIM, 128), jnp.float32),
        pltpu.VMEM((_DIM, 128), jnp.float32),
        pltpu.SemaphoreType.DMA,
        pltpu.SemaphoreType.DMA,
    ],
)
def _repack(w_hbm, wtail_hbm, wlin_hbm, ring, tailbuf, wt, sem_in, sem_out):
    wid = lax.axis_index("s") * _NC + lax.axis_index("c")

    def body(jj, carry):
        j = wid + jj * _NW

        @pl.when(j < _NFULL)
        def _():
            buf = ring.at[jj % 3]
            @pl.when(jj >= 3)
            def _():
                pltpu.make_async_copy(
                    ring.at[0], wlin_hbm.at[pl.ds(0, 16 * _DIM), :], sem_out
                ).wait()

            c0 = pl.multiple_of(j * _LC, _LC)
            for t in range(16):
                pltpu.async_copy(
                    w_hbm.at[:, pl.ds(c0 + t * 128, 128)],
                    buf.at[pl.ds(t * _DIM, _DIM), :],
                    sem_in,
                )
            pltpu.make_async_copy(
                wlin_hbm.at[pl.ds(0, 16 * _DIM), :], ring.at[0], sem_in
            ).wait()
            pltpu.async_copy(
                buf, wlin_hbm.at[pl.ds(j * 16 * _DIM, 16 * _DIM), :], sem_out
            )

        return carry

    nfj = _NFULL // _NW + 1
    lax.fori_loop(0, nfj, body, 0)
    for _ in range(3):
        pltpu.make_async_copy(
            ring.at[0], wlin_hbm.at[pl.ds(0, 16 * _DIM), :], sem_out
        ).wait()

    @pl.when(wid == _NFULL % _NW)
    def _():
        c0 = _NFULL * _LC
        for t in range(4):
            pltpu.async_copy(
                w_hbm.at[:, pl.ds(c0 + t * 128, 128)],
                tailbuf.at[pl.ds(t * _DIM, _DIM), :],
                sem_in,
            )
        pltpu.make_async_copy(
            wlin_hbm.at[pl.ds(0, 4 * _DIM), :], tailbuf, sem_in
        ).wait()
        pltpu.sync_copy(
            tailbuf, wlin_hbm.at[pl.ds(_NFULL * 16 * _DIM, 4 * _DIM), :]
        )

    @pl.when(wid == 0)
    def _():
        pltpu.sync_copy(wtail_hbm, wt)
        pltpu.sync_copy(wt, wlin_hbm.at[pl.ds((_NTC - 1) * _DIM, _DIM), :])


@functools.partial(
    pl.kernel,
    mesh=_mesh,
    out_type=jax.ShapeDtypeStruct((_DIM, _BATCH), jnp.float32),
    scratch_types=[
        pltpu.VMEM((_BPW,), jnp.int32),
        pltpu.VMEM((_DIM, _BPW), jnp.int32),
        pltpu.VMEM((_DIM, _BPW), jnp.float32),
        pltpu.SemaphoreType.DMA,
    ],
)
def _lookup(wlin_hbm, x_hbm, out_hbm, xv, idxv, rowsd, sem):
    wid = lax.axis_index("s") * _NC + lax.axis_index("c")
    base = wid * _BPW
    pltpu.sync_copy(x_hbm.at[pl.ds(base, _BPW)], xv)

    def offsets(c, carry):
        vx = xv[pl.ds(c * _NS, _NS)]
        vt = (vx >> 7) * 2048 + (vx & 127)
        for d in range(_DIM):
            idxv[d, pl.ds(c * _NS, _NS)] = vt + ((d // 8) * 1024 + (d % 8) * 128)
        return carry

    lax.fori_loop(0, _BPW // _NS, offsets, 0)

    copies = [
        pltpu.async_copy(
            wlin_hbm.at[idxv.at[d, pl.ds(k * _CHUNK, _CHUNK)]],
            rowsd.at[d, pl.ds(k * _CHUNK, _CHUNK)],
            sem,
        )
        for d in range(_DIM)
        for k in range(_NCH)
    ]
    for cp in copies:
        cp.wait()

    pltpu.sync_copy(rowsd, out_hbm.at[:, pl.ds(base, _BPW)])


def kernel(x, W):
    wtail = jnp.pad(W[:, _ALIGNED_END:], ((0, 0), (0, 128 - _RAG)))
    wlin = _repack(W, wtail).reshape(-1)
    h = _lookup(wlin, x.astype(jnp.int32))
    return h.T
